# Initial kernel scaffold; baseline (speedup 1.0000x reference)
import jax
import jax.numpy as jnp
from jax.experimental import pallas as pl


def kernel(protein_pos, protein_atom_feature, ligand_pos, ligand_atom_feature, batch_protein, batch_ligand, output_kind, Wp, bp, Wl, bl, eW1, eb1, eW2, eb2, infW, infb, nW1, nb1, nW2, nb2, oW1, ob1, oW2, ob2):
    raise NotImplementedError("write your pallas kernel here")


# speedup vs baseline: 3.9017x; 3.9017x over previous
import functools

import jax
import jax.numpy as jnp
from jax import lax
from jax.experimental import pallas as pl
from jax.experimental.pallas import tpu as pltpu
from jax.experimental.pallas import tpu_sc as plsc

NUM_LAYERS = 2
K = 32
NUM_G = 64
CUTOFF = 10.0
H = 128
B = 16

N_PAD = 10240
R_KNN = 64
CW = 512
T_KNN = N_PAD // R_KNN
R_E = 256
T_E = N_PAD // R_E

_GAP = CUTOFF / (NUM_G - 1)
_COEFF = -0.5 / (_GAP * _GAP)

_HIGH = jax.lax.Precision.HIGHEST


def _dot(a, b):
    return jnp.dot(a, b, precision=_HIGH, preferred_element_type=jnp.float32)


def _sc_gather(values, flat_idx, window=128):
    num0 = flat_idx.shape[0]
    cols = values.shape[1]
    num = ((num0 + window * 32 - 1) // (window * 32)) * (window * 32)
    if num != num0:
        flat_idx = jnp.concatenate(
            [flat_idx, jnp.zeros((num - num0,), flat_idx.dtype)])
    idx2 = flat_idx.reshape(1, num)
    mesh = plsc.VectorSubcoreMesh(core_axis_name="c", subcore_axis_name="s")

    @pl.kernel(
        out_type=jax.ShapeDtypeStruct((num, cols), values.dtype),
        mesh=mesh,
    )
    def gk(x_hbm, i_hbm, o_hbm):
        def body(i_vmem, o_vmem):
            pltpu.sync_copy(x_hbm.at[i_vmem.at[0]], o_vmem)

        pltpu.emit_pipeline(
            body,
            grid=(num // window,),
            in_specs=[pl.BlockSpec((1, window), lambda i: (0, i))],
            out_specs=[pl.BlockSpec((window, cols), lambda i: (i, 0))],
            core_axis_name=("c", "s"),
            dimension_semantics=(pltpu.PARALLEL,),
        )(i_hbm, o_hbm)

    out = gk(values, idx2)
    return out[:num0] if num != num0 else out


def _embed_kernel(f_ref, w_ref, o_ref):
    o_ref[...] = _dot(f_ref[...], w_ref[...])


def _embed(feat, w48):
    return pl.pallas_call(
        _embed_kernel,
        grid=(T_E,),
        in_specs=[
            pl.BlockSpec((R_E, H), lambda t: (t, 0)),
            pl.BlockSpec((H, H), lambda t: (0, 0)),
        ],
        out_specs=pl.BlockSpec((R_E, H), lambda t: (t, 0)),
        out_shape=jax.ShapeDtypeStruct((N_PAD, H), jnp.float32),
    )(feat, w48)


def _proj(h, wb):
    return pl.pallas_call(
        _embed_kernel,
        grid=(T_E,),
        in_specs=[
            pl.BlockSpec((R_E, H), lambda t: (t, 0)),
            pl.BlockSpec((H, H), lambda t: (0, 0)),
        ],
        out_specs=pl.BlockSpec((R_E, H), lambda t: (t, 0)),
        out_shape=jax.ShapeDtypeStruct((N_PAD, H), jnp.float32),
    )(h, wb)


def _knn_kernel(clo_ref, nch_ref, posr_ref, post_ref, rlo_ref, rhi_ref,
                idx_ref, d_ref, buf_ref):
    t = pl.program_id(0)
    clo = clo_ref[t]
    nch = nch_ref[t]
    rows = t * R_KNN + lax.broadcasted_iota(jnp.int32, (R_KNN, 1), 0)
    rlo = rlo_ref[...]
    rhi = rhi_ref[...]
    lane = lax.broadcasted_iota(jnp.int32, (1, CW), 1)
    inf = jnp.float32(jnp.inf)

    def fill(j, _):
        base = pl.multiple_of(clo + j * CW, CW)
        cid = base + lane
        acc = jnp.zeros((R_KNN, CW), jnp.float32)
        for c in range(3):
            diff = posr_ref[:, c:c + 1] - post_ref[c:c + 1, pl.ds(base, CW)]
            acc = acc + diff * diff
        masked = (cid < rlo) | (cid >= rhi) | (cid == rows)
        buf_ref[:, pl.ds(base, CW)] = jnp.where(masked, inf, acc)
        return 0

    lax.fori_loop(0, nch, fill, 0)

    prev = None
    for k in range(K):
        def chunk(j, carry, prev=prev):
            val, vidx = carry
            base = pl.multiple_of(clo + j * CW, CW)
            cid = base + lane
            c = buf_ref[:, pl.ds(base, CW)]
            if prev is not None:
                c = jnp.where(cid == prev, inf, c)
                buf_ref[:, pl.ds(base, CW)] = c
            mval = jnp.min(c, axis=1, keepdims=True)
            midx = jnp.min(jnp.where(c == mval, cid, N_PAD), axis=1,
                           keepdims=True)
            upd = mval < val
            return (jnp.where(upd, mval, val), jnp.where(upd, midx, vidx))

        val, vidx = lax.fori_loop(
            0, nch, chunk,
            (jnp.full((R_KNN, 1), inf), jnp.zeros((R_KNN, 1), jnp.int32)))
        idx_ref[k, :, :] = vidx
        d_ref[k, :, :] = jnp.sqrt(val)
        prev = vidx


def _knn(clo, nch, pos_rows, pos_t, rlo, rhi):
    return pl.pallas_call(
        _knn_kernel,
        grid=(T_KNN,),
        in_specs=[
            pl.BlockSpec(memory_space=pltpu.SMEM),
            pl.BlockSpec(memory_space=pltpu.SMEM),
            pl.BlockSpec((R_KNN, 8), lambda t: (t, 0)),
            pl.BlockSpec((8, N_PAD), lambda t: (0, 0)),
            pl.BlockSpec((R_KNN, 1), lambda t: (t, 0)),
            pl.BlockSpec((R_KNN, 1), lambda t: (t, 0)),
        ],
        out_specs=[
            pl.BlockSpec((K, R_KNN, 1), lambda t: (0, t, 0)),
            pl.BlockSpec((K, R_KNN, 1), lambda t: (0, t, 0)),
        ],
        out_shape=[
            jax.ShapeDtypeStruct((K, N_PAD, 1), jnp.int32),
            jax.ShapeDtypeStruct((K, N_PAD, 1), jnp.float32),
        ],
        scratch_shapes=[pltpu.VMEM((R_KNN, N_PAD), jnp.float32)],
    )(clo, nch, pos_rows, pos_t, rlo, rhi)


def _layer_kernel(h_ref, d_ref, g_ref, wg_ref, wa_ref, eb1_ref, ew2_ref,
                  eb2_ref, infw_ref, infb_ref, nw1a_ref, nw1b_ref, nb1_ref,
                  nw2_ref, nb2_ref, o_ref):
    h = h_ref[...]
    a = _dot(h, wa_ref[...]) + eb1_ref[...]
    offs = lax.broadcasted_iota(jnp.int32, (1, NUM_G), 1).astype(
        jnp.float32) * jnp.float32(_GAP)
    eb2 = eb2_ref[...]
    infw = infw_ref[...]
    infb = infb_ref[0:1, 0:1]
    wg = wg_ref[...]
    ew2 = ew2_ref[...]

    def body(k, mi):
        bk = g_ref[pl.ds(k, 1), :, :].reshape(R_E, H)
        dk = d_ref[pl.ds(k, 1), :, :].reshape(R_E, 1)
        rbf = jnp.exp(jnp.float32(_COEFF) * (dk - offs) ** 2)
        m1 = jnp.maximum(_dot(rbf, wg) + a + bk, 0.0)
        m2 = jnp.maximum(_dot(m1, ew2) + eb2, 0.0)
        s = jnp.sum(m2 * infw, axis=-1, keepdims=True)
        gate = jax.nn.sigmoid(s + infb)
        return mi + gate * m2

    mi = lax.fori_loop(0, K, body, jnp.zeros((R_E, H), jnp.float32))
    n1 = jnp.maximum(_dot(mi, nw1a_ref[...]) + _dot(h, nw1b_ref[...])
                     + nb1_ref[...], 0.0)
    o_ref[...] = h + _dot(n1, nw2_ref[...]) + nb2_ref[...]


def _layer(h, dkm, gr, wg, wa, eb1r, ew2, eb2r, infwr, infbr, nw1a, nw1b,
           nb1r, nw2, nb2r):
    full = lambda shape: pl.BlockSpec(shape, lambda t: tuple(0 for _ in shape))
    return pl.pallas_call(
        _layer_kernel,
        grid=(T_E,),
        in_specs=[
            pl.BlockSpec((R_E, H), lambda t: (t, 0)),
            pl.BlockSpec((K, R_E, 1), lambda t: (0, t, 0)),
            pl.BlockSpec((K, R_E, H), lambda t: (0, t, 0)),
            full((NUM_G, H)),
            full((H, H)),
            full((1, H)),
            full((H, H)),
            full((1, H)),
            full((1, H)),
            full((1, H)),
            full((H, H)),
            full((H, H)),
            full((1, H)),
            full((H, H)),
            full((1, H)),
        ],
        out_specs=pl.BlockSpec((R_E, H), lambda t: (t, 0)),
        out_shape=jax.ShapeDtypeStruct((N_PAD, H), jnp.float32),
    )(h, dkm, gr, wg, wa, eb1r, ew2, eb2r, infwr, infbr, nw1a, nw1b, nb1r,
      nw2, nb2r)


def _pool_kernel(h_ref, bt_ref, ow1_ref, ob1_ref, ow2_ref, ob2_ref, kind_ref,
                 o_ref):
    bt = bt_ref[...]
    gid = lax.broadcasted_iota(jnp.int32, (B, 1), 0).astype(jnp.float32)
    onehot = jnp.where(bt == gid, 1.0, 0.0)
    pre = _dot(onehot, h_ref[...])
    o = _dot(pre, ow1_ref[...]) + ob1_ref[...]
    o = jax.nn.softplus(o) - jnp.float32(jnp.log(2.0))
    o = _dot(o, ow2_ref[...]) + ob2_ref[...]
    kidx = lax.broadcasted_iota(jnp.int32, (B, 3), 1).astype(jnp.float32)
    mask = jnp.where(kidx == kind_ref[...] - 1.0, 1.0, 0.0)
    o_ref[...] = jnp.sum(o * mask, axis=-1, keepdims=True)


def _pool(h, batch_f, ow1, ob1r, ow2p, ob2r, kind_f):
    full = lambda shape: pl.BlockSpec(shape, lambda t: tuple(0 for _ in shape))
    return pl.pallas_call(
        _pool_kernel,
        grid=(1,),
        in_specs=[
            full((N_PAD, H)),
            full((1, N_PAD)),
            full((H, H)),
            full((1, H)),
            full((H, 3)),
            full((1, 3)),
            full((B, 1)),
        ],
        out_specs=full((B, 1)),
        out_shape=jax.ShapeDtypeStruct((B, 1), jnp.float32),
    )(h, batch_f, ow1, ob1r, ow2p, ob2r, kind_f)


def kernel(protein_pos, protein_atom_feature, ligand_pos, ligand_atom_feature,
           batch_protein, batch_ligand, output_kind,
           Wp, bp, Wl, bl, eW1, eb1, eW2, eb2, infW, infb,
           nW1, nb1, nW2, nb2, oW1, ob1, oW2, ob2):
    npn = protein_pos.shape[0]
    nl = ligand_pos.shape[0]
    n = npn + nl
    dp = protein_atom_feature.shape[1]
    dl = ligand_atom_feature.shape[1]

    br = jnp.arange(B + 1, dtype=jnp.int32)
    sp = jnp.searchsorted(batch_protein, br, side="left").astype(jnp.int32)
    sl = jnp.searchsorted(batch_ligand, br, side="left").astype(jnp.int32)
    off = sp + sl

    q = jnp.arange(N_PAD, dtype=jnp.int32)
    valid = q < n
    bq = jnp.clip(jnp.searchsorted(off, q, side="right").astype(jnp.int32) - 1,
                  0, B - 1)
    npb = sp[bq + 1] - sp[bq]
    rq = q - off[bq]
    perm = jnp.where(rq < npb, sp[bq] + rq, npn + sl[bq] + (rq - npb))
    perm = jnp.where(valid, perm, 0)
    row_lo = jnp.where(valid, off[bq], 0).reshape(N_PAD, 1)
    row_hi = jnp.where(valid, off[bq + 1], 0).reshape(N_PAD, 1)
    batch_f = jnp.where(valid, bq, B).astype(jnp.float32).reshape(1, N_PAD)

    tstart = jnp.arange(T_KNN, dtype=jnp.int32) * R_KNN
    tlast = jnp.minimum(tstart + R_KNN - 1, n - 1)
    bfirst = bq[tstart]
    blast = bq[tlast]
    clo = (off[bfirst] // CW) * CW
    chi = ((off[blast + 1] + CW - 1) // CW) * CW
    nch = jnp.where(tstart < n, (chi - clo) // CW, 0)
    clo = jnp.where(tstart < n, clo, 0)

    zcol = lambda r, c: jnp.zeros((r, c), jnp.float32)
    prot = jnp.concatenate(
        [protein_atom_feature, zcol(npn, 40 - dp), jnp.ones((npn, 1)),
         zcol(npn, 1), zcol(npn, 6), protein_pos, zcol(npn, 77)], axis=1)
    lig = jnp.concatenate(
        [zcol(nl, dp), ligand_atom_feature, zcol(nl, 40 - dp - dl),
         zcol(nl, 1), jnp.ones((nl, 1)), zcol(nl, 6), ligand_pos,
         zcol(nl, 77)], axis=1)
    fpcomb = jnp.concatenate([prot, lig], axis=0)
    w128 = jnp.concatenate(
        [Wp, Wl, bp[None, :], bl[None, :], jnp.zeros((H - 42, H))],
        axis=0).astype(jnp.float32)

    sortedfp = _sc_gather(fpcomb, perm)
    posm = sortedfp[:, 48:56]
    pos_t = posm.T

    h = _embed(sortedfp, w128)

    idxkm, dkm = _knn(clo, nch, posm, pos_t, row_lo, row_hi)
    idx_flat = idxkm.reshape(K * N_PAD)

    for l in range(NUM_LAYERS):
        bv = _proj(h, eW1[l, NUM_G + H:, :])
        g = _sc_gather(bv, idx_flat)
        gr = g.reshape(K, N_PAD, H)
        h = _layer(
            h, dkm, gr,
            eW1[l, :NUM_G, :], eW1[l, NUM_G:NUM_G + H, :],
            eb1[l].reshape(1, H), eW2[l], eb2[l].reshape(1, H),
            infW[l, :, 0].reshape(1, H),
            jnp.broadcast_to(infb[l].reshape(1, 1), (1, H)),
            nW1[l, :H, :], nW1[l, H:, :], nb1[l].reshape(1, H),
            nW2[l], nb2[l].reshape(1, H))

    return _pool(h, batch_f, oW1, ob1.reshape(1, H), oW2, ob2.reshape(1, 3),
                 output_kind.astype(jnp.float32).reshape(B, 1))


# speedup vs baseline: 6.0406x; 1.5482x over previous
import functools

import jax
import jax.numpy as jnp
from jax import lax
from jax.experimental import pallas as pl
from jax.experimental.pallas import tpu as pltpu
from jax.experimental.pallas import tpu_sc as plsc

NUM_LAYERS = 2
K = 32
NUM_G = 64
CUTOFF = 10.0
H = 128
B = 16

N_PAD = 10240
R_KNN = 128
CW = 256
T_KNN = N_PAD // R_KNN
R_E = 256
T_E = N_PAD // R_E

_GAP = CUTOFF / (NUM_G - 1)
_COEFF = -0.5 / (_GAP * _GAP)

_HIGH = jax.lax.Precision.HIGHEST


def _dot(a, b):
    return jnp.dot(a, b, precision=_HIGH, preferred_element_type=jnp.float32)


def _sc_gather(values, flat_idx, window=128):
    num0 = flat_idx.shape[0]
    cols = values.shape[1]
    num = ((num0 + window * 32 - 1) // (window * 32)) * (window * 32)
    if num != num0:
        flat_idx = jnp.concatenate(
            [flat_idx, jnp.zeros((num - num0,), flat_idx.dtype)])
    idx2 = flat_idx.reshape(1, num)
    mesh = plsc.VectorSubcoreMesh(core_axis_name="c", subcore_axis_name="s")

    @pl.kernel(
        out_type=jax.ShapeDtypeStruct((num, cols), values.dtype),
        mesh=mesh,
    )
    def gk(x_hbm, i_hbm, o_hbm):
        def body(i_vmem, o_vmem):
            pltpu.sync_copy(x_hbm.at[i_vmem.at[0]], o_vmem)

        pltpu.emit_pipeline(
            body,
            grid=(num // window,),
            in_specs=[pl.BlockSpec((1, window), lambda i: (0, i))],
            out_specs=[pl.BlockSpec((window, cols), lambda i: (i, 0))],
            core_axis_name=("c", "s"),
            dimension_semantics=(pltpu.PARALLEL,),
        )(i_hbm, o_hbm)

    out = gk(values, idx2)
    return out[:num0] if num != num0 else out


def _embed_kernel(f_ref, w_ref, o_ref):
    o_ref[...] = _dot(f_ref[...], w_ref[...])


def _embed(feat, w48):
    return pl.pallas_call(
        _embed_kernel,
        grid=(T_E,),
        in_specs=[
            pl.BlockSpec((R_E, H), lambda t: (t, 0)),
            pl.BlockSpec((H, H), lambda t: (0, 0)),
        ],
        out_specs=pl.BlockSpec((R_E, H), lambda t: (t, 0)),
        out_shape=jax.ShapeDtypeStruct((N_PAD, H), jnp.float32),
    )(feat, w48)


def _proj(h, wb):
    return pl.pallas_call(
        _embed_kernel,
        grid=(T_E,),
        in_specs=[
            pl.BlockSpec((R_E, H), lambda t: (t, 0)),
            pl.BlockSpec((H, H), lambda t: (0, 0)),
        ],
        out_specs=pl.BlockSpec((R_E, H), lambda t: (t, 0)),
        out_shape=jax.ShapeDtypeStruct((N_PAD, H), jnp.float32),
    )(h, wb)


def _knn_kernel(clo_ref, nch_ref, posm_ref, post_ref, rlo_ref, rhi_ref,
                idx_ref, d_ref, buf_ref):
    t = pl.program_id(0)
    clo = clo_ref[t]
    nch = nch_ref[t]
    rows = t * R_KNN + lax.broadcasted_iota(jnp.int32, (1, R_KNN), 1)
    rlo = rlo_ref[...]
    rhi = rhi_ref[...]
    sub = lax.broadcasted_iota(jnp.int32, (CW, 1), 0)
    inf = jnp.float32(jnp.inf)
    ptile = post_ref[...]
    sqr = jnp.sum(ptile * ptile, axis=0, keepdims=True)

    def fill(j, _):
        base = pl.multiple_of(clo + j * CW, CW)
        cid = base + sub
        pc = posm_ref[pl.ds(base, CW), :]
        crs = jnp.dot(pc.astype(jnp.bfloat16), ptile.astype(jnp.bfloat16),
                      preferred_element_type=jnp.float32)
        sqc = jnp.sum(pc * pc, axis=1, keepdims=True)
        d2 = (sqc + sqr) - 2.0 * crs
        masked = (cid < rlo) | (cid >= rhi) | (cid == rows)
        buf_ref[pl.ds(base, CW), :] = jnp.where(masked, inf, d2)
        return 0

    lax.fori_loop(0, nch, fill, 0)

    prev = None
    for k in range(K):
        def chunk(j, carry, prev=prev):
            val, vidx = carry
            base = pl.multiple_of(clo + j * CW, CW)
            cid = base + sub
            c = buf_ref[pl.ds(base, CW), :]
            if prev is not None:
                c = jnp.where(cid == prev, inf, c)
                buf_ref[pl.ds(base, CW), :] = c
            mval = jnp.min(c, axis=0, keepdims=True)
            midx = jnp.min(jnp.where(c == mval, cid, N_PAD), axis=0,
                           keepdims=True)
            upd = mval < val
            return (jnp.where(upd, mval, val), jnp.where(upd, midx, vidx))

        val, vidx = lax.fori_loop(
            0, nch, chunk,
            (jnp.full((1, R_KNN), inf), jnp.zeros((1, R_KNN), jnp.int32)))
        idx_ref[k:k + 1, :] = vidx
        d_ref[k:k + 1, :] = jnp.sqrt(val)
        prev = vidx


def _knn(clo, nch, posm, pos_t, rlo, rhi):
    return pl.pallas_call(
        _knn_kernel,
        grid=(T_KNN,),
        in_specs=[
            pl.BlockSpec(memory_space=pltpu.SMEM),
            pl.BlockSpec(memory_space=pltpu.SMEM),
            pl.BlockSpec((N_PAD, 8), lambda t: (0, 0)),
            pl.BlockSpec((8, R_KNN), lambda t: (0, t)),
            pl.BlockSpec((1, R_KNN), lambda t: (0, t)),
            pl.BlockSpec((1, R_KNN), lambda t: (0, t)),
        ],
        out_specs=[
            pl.BlockSpec((K, R_KNN), lambda t: (0, t)),
            pl.BlockSpec((K, R_KNN), lambda t: (0, t)),
        ],
        out_shape=[
            jax.ShapeDtypeStruct((K, N_PAD), jnp.int32),
            jax.ShapeDtypeStruct((K, N_PAD), jnp.float32),
        ],
        scratch_shapes=[pltpu.VMEM((N_PAD, R_KNN), jnp.float32)],
    )(clo, nch, posm, pos_t, rlo, rhi)


def _layer_kernel(h_ref, d_ref, g_ref, wg_ref, wa_ref, eb1_ref, ew2_ref,
                  eb2_ref, infw_ref, infb_ref, nw1a_ref, nw1b_ref, nb1_ref,
                  nw2_ref, nb2_ref, o_ref):
    h = h_ref[...]
    a = _dot(h, wa_ref[...]) + eb1_ref[...]
    offs = lax.broadcasted_iota(jnp.int32, (1, NUM_G), 1).astype(
        jnp.float32) * jnp.float32(_GAP)
    eb2 = eb2_ref[...]
    infw = infw_ref[...]
    infb = infb_ref[0:1, 0:1]
    wg = wg_ref[...]
    ew2 = ew2_ref[...]

    def body(k, mi):
        bk = g_ref[pl.ds(k, 1), :, :].reshape(R_E, H)
        dk = d_ref[pl.ds(k, 1), :, :].reshape(R_E, 1)
        rbf = jnp.exp(jnp.float32(_COEFF) * (dk - offs) ** 2)
        m1 = jnp.maximum(_dot(rbf, wg) + a + bk, 0.0)
        m2 = jnp.maximum(_dot(m1, ew2) + eb2, 0.0)
        s = jnp.sum(m2 * infw, axis=-1, keepdims=True)
        gate = jax.nn.sigmoid(s + infb)
        return mi + gate * m2

    mi = lax.fori_loop(0, K, body, jnp.zeros((R_E, H), jnp.float32))
    n1 = jnp.maximum(_dot(mi, nw1a_ref[...]) + _dot(h, nw1b_ref[...])
                     + nb1_ref[...], 0.0)
    o_ref[...] = h + _dot(n1, nw2_ref[...]) + nb2_ref[...]


def _layer(h, dkm, gr, wg, wa, eb1r, ew2, eb2r, infwr, infbr, nw1a, nw1b,
           nb1r, nw2, nb2r):
    full = lambda shape: pl.BlockSpec(shape, lambda t: tuple(0 for _ in shape))
    return pl.pallas_call(
        _layer_kernel,
        grid=(T_E,),
        in_specs=[
            pl.BlockSpec((R_E, H), lambda t: (t, 0)),
            pl.BlockSpec((K, R_E, 1), lambda t: (0, t, 0)),
            pl.BlockSpec((K, R_E, H), lambda t: (0, t, 0)),
            full((NUM_G, H)),
            full((H, H)),
            full((1, H)),
            full((H, H)),
            full((1, H)),
            full((1, H)),
            full((1, H)),
            full((H, H)),
            full((H, H)),
            full((1, H)),
            full((H, H)),
            full((1, H)),
        ],
        out_specs=pl.BlockSpec((R_E, H), lambda t: (t, 0)),
        out_shape=jax.ShapeDtypeStruct((N_PAD, H), jnp.float32),
    )(h, dkm, gr, wg, wa, eb1r, ew2, eb2r, infwr, infbr, nw1a, nw1b, nb1r,
      nw2, nb2r)


def _pool_kernel(h_ref, bt_ref, ow1_ref, ob1_ref, ow2_ref, ob2_ref, kind_ref,
                 o_ref):
    bt = bt_ref[...]
    gid = lax.broadcasted_iota(jnp.int32, (B, 1), 0).astype(jnp.float32)
    onehot = jnp.where(bt == gid, 1.0, 0.0)
    pre = _dot(onehot, h_ref[...])
    o = _dot(pre, ow1_ref[...]) + ob1_ref[...]
    o = jax.nn.softplus(o) - jnp.float32(jnp.log(2.0))
    o = _dot(o, ow2_ref[...]) + ob2_ref[...]
    kidx = lax.broadcasted_iota(jnp.int32, (B, 3), 1).astype(jnp.float32)
    mask = jnp.where(kidx == kind_ref[...] - 1.0, 1.0, 0.0)
    o_ref[...] = jnp.sum(o * mask, axis=-1, keepdims=True)


def _pool(h, batch_f, ow1, ob1r, ow2p, ob2r, kind_f):
    full = lambda shape: pl.BlockSpec(shape, lambda t: tuple(0 for _ in shape))
    return pl.pallas_call(
        _pool_kernel,
        grid=(1,),
        in_specs=[
            full((N_PAD, H)),
            full((1, N_PAD)),
            full((H, H)),
            full((1, H)),
            full((H, 3)),
            full((1, 3)),
            full((B, 1)),
        ],
        out_specs=full((B, 1)),
        out_shape=jax.ShapeDtypeStruct((B, 1), jnp.float32),
    )(h, batch_f, ow1, ob1r, ow2p, ob2r, kind_f)


def kernel(protein_pos, protein_atom_feature, ligand_pos, ligand_atom_feature,
           batch_protein, batch_ligand, output_kind,
           Wp, bp, Wl, bl, eW1, eb1, eW2, eb2, infW, infb,
           nW1, nb1, nW2, nb2, oW1, ob1, oW2, ob2):
    npn = protein_pos.shape[0]
    nl = ligand_pos.shape[0]
    n = npn + nl
    dp = protein_atom_feature.shape[1]
    dl = ligand_atom_feature.shape[1]

    br = jnp.arange(B + 1, dtype=jnp.int32)
    sp = jnp.searchsorted(batch_protein, br, side="left").astype(jnp.int32)
    sl = jnp.searchsorted(batch_ligand, br, side="left").astype(jnp.int32)
    off = sp + sl

    q = jnp.arange(N_PAD, dtype=jnp.int32)
    valid = q < n
    bq = jnp.clip(jnp.searchsorted(off, q, side="right").astype(jnp.int32) - 1,
                  0, B - 1)
    npb = sp[bq + 1] - sp[bq]
    rq = q - off[bq]
    perm = jnp.where(rq < npb, sp[bq] + rq, npn + sl[bq] + (rq - npb))
    perm = jnp.where(valid, perm, 0)
    row_lo = jnp.where(valid, off[bq], 0).reshape(1, N_PAD)
    row_hi = jnp.where(valid, off[bq + 1], 0).reshape(1, N_PAD)
    batch_f = jnp.where(valid, bq, B).astype(jnp.float32).reshape(1, N_PAD)

    tstart = jnp.arange(T_KNN, dtype=jnp.int32) * R_KNN
    tlast = jnp.minimum(tstart + R_KNN - 1, n - 1)
    bfirst = bq[tstart]
    blast = bq[tlast]
    clo = (off[bfirst] // CW) * CW
    chi = ((off[blast + 1] + CW - 1) // CW) * CW
    nch = jnp.where(tstart < n, (chi - clo) // CW, 0)
    clo = jnp.where(tstart < n, clo, 0)

    zcol = lambda r, c: jnp.zeros((r, c), jnp.float32)
    prot = jnp.concatenate(
        [protein_atom_feature, zcol(npn, 40 - dp), jnp.ones((npn, 1)),
         zcol(npn, 1), zcol(npn, 6), protein_pos, zcol(npn, 77)], axis=1)
    lig = jnp.concatenate(
        [zcol(nl, dp), ligand_atom_feature, zcol(nl, 40 - dp - dl),
         zcol(nl, 1), jnp.ones((nl, 1)), zcol(nl, 6), ligand_pos,
         zcol(nl, 77)], axis=1)
    fpcomb = jnp.concatenate([prot, lig], axis=0)
    w128 = jnp.concatenate(
        [Wp, Wl, bp[None, :], bl[None, :], jnp.zeros((H - 42, H))],
        axis=0).astype(jnp.float32)

    sortedfp = _sc_gather(fpcomb, perm)
    posm = sortedfp[:, 48:56]
    pos_t = posm.T

    h = _embed(sortedfp, w128)

    idxkm, dkm = _knn(clo, nch, posm, pos_t, row_lo, row_hi)
    idx_flat = idxkm.reshape(K * N_PAD)
    d3 = dkm[:, :, None]

    for l in range(NUM_LAYERS):
        bv = _proj(h, eW1[l, NUM_G + H:, :])
        g = _sc_gather(bv, idx_flat)
        gr = g.reshape(K, N_PAD, H)
        h = _layer(
            h, d3, gr,
            eW1[l, :NUM_G, :], eW1[l, NUM_G:NUM_G + H, :],
            eb1[l].reshape(1, H), eW2[l], eb2[l].reshape(1, H),
            infW[l, :, 0].reshape(1, H),
            jnp.broadcast_to(infb[l].reshape(1, 1), (1, H)),
            nW1[l, :H, :], nW1[l, H:, :], nb1[l].reshape(1, H),
            nW2[l], nb2[l].reshape(1, H))

    return _pool(h, batch_f, oW1, ob1.reshape(1, H), oW2, ob2.reshape(1, 3),
                 output_kind.astype(jnp.float32).reshape(B, 1))


# speedup vs baseline: 6.6711x; 1.1044x over previous
import functools

import jax
import jax.numpy as jnp
from jax import lax
from jax.experimental import pallas as pl
from jax.experimental.pallas import tpu as pltpu
from jax.experimental.pallas import tpu_sc as plsc

NUM_LAYERS = 2
K = 32
NUM_G = 64
CUTOFF = 10.0
H = 128
B = 16

N_PAD = 10240
R_KNN = 128
CW = 256
T_KNN = N_PAD // R_KNN
R_E = 256
T_E = N_PAD // R_E

_GAP = CUTOFF / (NUM_G - 1)
_COEFF = -0.5 / (_GAP * _GAP)

_HIGH = jax.lax.Precision.HIGHEST


def _dot(a, b):
    return jnp.dot(a, b, precision=_HIGH, preferred_element_type=jnp.float32)


def _dot3(a, b):
    ah = a.astype(jnp.bfloat16)
    al = (a - ah.astype(jnp.float32)).astype(jnp.bfloat16)
    bh = b.astype(jnp.bfloat16)
    bl = (b - bh.astype(jnp.float32)).astype(jnp.bfloat16)
    f = jnp.float32
    d = lambda x, y: jnp.dot(x, y, preferred_element_type=f)
    return d(ah, bh) + (d(al, bh) + d(ah, bl))


def _sc_gather(values, flat_idx, window=256):
    num0 = flat_idx.shape[0]
    cols = values.shape[1]
    num = ((num0 + window * 32 - 1) // (window * 32)) * (window * 32)
    if num != num0:
        flat_idx = jnp.concatenate(
            [flat_idx, jnp.zeros((num - num0,), flat_idx.dtype)])
    idx2 = flat_idx.reshape(1, num)
    mesh = plsc.VectorSubcoreMesh(core_axis_name="c", subcore_axis_name="s")

    @pl.kernel(
        out_type=jax.ShapeDtypeStruct((num, cols), values.dtype),
        mesh=mesh,
    )
    def gk(x_hbm, i_hbm, o_hbm):
        def body(i_vmem, o_vmem):
            pltpu.sync_copy(x_hbm.at[i_vmem.at[0]], o_vmem)

        pltpu.emit_pipeline(
            body,
            grid=(num // window,),
            in_specs=[pl.BlockSpec((1, window), lambda i: (0, i))],
            out_specs=[pl.BlockSpec((window, cols), lambda i: (i, 0))],
            core_axis_name=("c", "s"),
            dimension_semantics=(pltpu.PARALLEL,),
        )(i_hbm, o_hbm)

    out = gk(values, idx2)
    return out[:num0] if num != num0 else out


def _embed_kernel(f_ref, w_ref, o_ref):
    o_ref[...] = _dot(f_ref[...], w_ref[...])


def _embed(feat, w48):
    return pl.pallas_call(
        _embed_kernel,
        grid=(T_E,),
        in_specs=[
            pl.BlockSpec((R_E, H), lambda t: (t, 0)),
            pl.BlockSpec((H, H), lambda t: (0, 0)),
        ],
        out_specs=pl.BlockSpec((R_E, H), lambda t: (t, 0)),
        out_shape=jax.ShapeDtypeStruct((N_PAD, H), jnp.float32),
    )(feat, w48)


def _proj(h, wb):
    return pl.pallas_call(
        _embed_kernel,
        grid=(T_E,),
        in_specs=[
            pl.BlockSpec((R_E, H), lambda t: (t, 0)),
            pl.BlockSpec((H, H), lambda t: (0, 0)),
        ],
        out_specs=pl.BlockSpec((R_E, H), lambda t: (t, 0)),
        out_shape=jax.ShapeDtypeStruct((N_PAD, H), jnp.float32),
    )(h, wb)


def _knn_kernel(clo_ref, nch_ref, posm_ref, post_ref, rlo_ref, rhi_ref,
                idx_ref, d_ref, buf_ref):
    t = pl.program_id(0)
    clo = clo_ref[t]
    nch = nch_ref[t]
    rows = t * R_KNN + lax.broadcasted_iota(jnp.int32, (1, R_KNN), 1)
    rlo = rlo_ref[...]
    rhi = rhi_ref[...]
    sub = lax.broadcasted_iota(jnp.int32, (CW, 1), 0)
    inf = jnp.float32(jnp.inf)
    ptile = post_ref[...]
    sqr = jnp.sum(ptile * ptile, axis=0, keepdims=True)

    def fill(j, _):
        base = pl.multiple_of(clo + j * CW, CW)
        cid = base + sub
        pc = posm_ref[pl.ds(base, CW), :]
        crs = jnp.dot(pc.astype(jnp.bfloat16), ptile.astype(jnp.bfloat16),
                      preferred_element_type=jnp.float32)
        sqc = jnp.sum(pc * pc, axis=1, keepdims=True)
        d2 = (sqc + sqr) - 2.0 * crs
        masked = (cid < rlo) | (cid >= rhi) | (cid == rows)
        buf_ref[pl.ds(base, CW), :] = jnp.where(masked, inf, d2)
        return 0

    lax.fori_loop(0, nch, fill, 0)

    prev = None
    for k in range(K):
        def chunk(j, carry, prev=prev):
            val, vidx = carry
            base = pl.multiple_of(clo + j * CW, CW)
            cid = base + sub
            c = buf_ref[pl.ds(base, CW), :]
            if prev is not None:
                c = jnp.where(cid == prev, inf, c)
                buf_ref[pl.ds(base, CW), :] = c
            mval = jnp.min(c, axis=0, keepdims=True)
            midx = jnp.min(jnp.where(c == mval, cid, N_PAD), axis=0,
                           keepdims=True)
            upd = mval < val
            return (jnp.where(upd, mval, val), jnp.where(upd, midx, vidx))

        val, vidx = lax.fori_loop(
            0, nch, chunk,
            (jnp.full((1, R_KNN), inf), jnp.zeros((1, R_KNN), jnp.int32)))
        idx_ref[k:k + 1, :] = vidx
        d_ref[k:k + 1, :] = jnp.sqrt(val)
        prev = vidx


def _knn(clo, nch, posm, pos_t, rlo, rhi):
    return pl.pallas_call(
        _knn_kernel,
        grid=(T_KNN,),
        in_specs=[
            pl.BlockSpec(memory_space=pltpu.SMEM),
            pl.BlockSpec(memory_space=pltpu.SMEM),
            pl.BlockSpec((N_PAD, 8), lambda t: (0, 0)),
            pl.BlockSpec((8, R_KNN), lambda t: (0, t)),
            pl.BlockSpec((1, R_KNN), lambda t: (0, t)),
            pl.BlockSpec((1, R_KNN), lambda t: (0, t)),
        ],
        out_specs=[
            pl.BlockSpec((K, R_KNN), lambda t: (0, t)),
            pl.BlockSpec((K, R_KNN), lambda t: (0, t)),
        ],
        out_shape=[
            jax.ShapeDtypeStruct((K, N_PAD), jnp.int32),
            jax.ShapeDtypeStruct((K, N_PAD), jnp.float32),
        ],
        scratch_shapes=[pltpu.VMEM((N_PAD, R_KNN), jnp.float32)],
    )(clo, nch, posm, pos_t, rlo, rhi)


def _layer_kernel(h_ref, d_ref, g_ref, wg_ref, wa_ref, eb1_ref, ew2_ref,
                  eb2_ref, infw_ref, infb_ref, nw1a_ref, nw1b_ref, nb1_ref,
                  nw2_ref, nb2_ref, o_ref):
    h = h_ref[...]
    a = _dot(h, wa_ref[...]) + eb1_ref[...]
    offs = lax.broadcasted_iota(jnp.int32, (1, NUM_G), 1).astype(
        jnp.float32) * jnp.float32(_GAP)
    eb2 = eb2_ref[...]
    infw = infw_ref[...]
    infb = infb_ref[0:1, 0:1]
    wg = wg_ref[...]
    ew2 = ew2_ref[...]

    def body(k, mi):
        bk = g_ref[pl.ds(k, 1), :, :].reshape(R_E, H)
        dk = d_ref[pl.ds(k, 1), :, :].reshape(R_E, 1)
        rbf = jnp.exp(jnp.float32(_COEFF) * (dk - offs) ** 2)
        m1 = jnp.maximum(_dot3(rbf, wg) + a + bk, 0.0)
        m2 = jnp.maximum(_dot3(m1, ew2) + eb2, 0.0)
        s = jnp.sum(m2 * infw, axis=-1, keepdims=True)
        gate = jax.nn.sigmoid(s + infb)
        return mi + gate * m2

    mi = lax.fori_loop(0, K, body, jnp.zeros((R_E, H), jnp.float32))
    n1 = jnp.maximum(_dot(mi, nw1a_ref[...]) + _dot(h, nw1b_ref[...])
                     + nb1_ref[...], 0.0)
    o_ref[...] = h + _dot(n1, nw2_ref[...]) + nb2_ref[...]


def _layer(h, dkm, gr, wg, wa, eb1r, ew2, eb2r, infwr, infbr, nw1a, nw1b,
           nb1r, nw2, nb2r):
    full = lambda shape: pl.BlockSpec(shape, lambda t: tuple(0 for _ in shape))
    return pl.pallas_call(
        _layer_kernel,
        grid=(T_E,),
        in_specs=[
            pl.BlockSpec((R_E, H), lambda t: (t, 0)),
            pl.BlockSpec((K, R_E, 1), lambda t: (0, t, 0)),
            pl.BlockSpec((K, R_E, H), lambda t: (0, t, 0)),
            full((NUM_G, H)),
            full((H, H)),
            full((1, H)),
            full((H, H)),
            full((1, H)),
            full((1, H)),
            full((1, H)),
            full((H, H)),
            full((H, H)),
            full((1, H)),
            full((H, H)),
            full((1, H)),
        ],
        out_specs=pl.BlockSpec((R_E, H), lambda t: (t, 0)),
        out_shape=jax.ShapeDtypeStruct((N_PAD, H), jnp.float32),
    )(h, dkm, gr, wg, wa, eb1r, ew2, eb2r, infwr, infbr, nw1a, nw1b, nb1r,
      nw2, nb2r)


def _pool_kernel(h_ref, bt_ref, ow1_ref, ob1_ref, ow2_ref, ob2_ref, kind_ref,
                 o_ref):
    bt = bt_ref[...]
    gid = lax.broadcasted_iota(jnp.int32, (B, 1), 0).astype(jnp.float32)
    onehot = jnp.where(bt == gid, 1.0, 0.0)
    pre = _dot(onehot, h_ref[...])
    o = _dot(pre, ow1_ref[...]) + ob1_ref[...]
    o = jax.nn.softplus(o) - jnp.float32(jnp.log(2.0))
    o = _dot(o, ow2_ref[...]) + ob2_ref[...]
    kidx = lax.broadcasted_iota(jnp.int32, (B, 3), 1).astype(jnp.float32)
    mask = jnp.where(kidx == kind_ref[...] - 1.0, 1.0, 0.0)
    o_ref[...] = jnp.sum(o * mask, axis=-1, keepdims=True)


def _pool(h, batch_f, ow1, ob1r, ow2p, ob2r, kind_f):
    full = lambda shape: pl.BlockSpec(shape, lambda t: tuple(0 for _ in shape))
    return pl.pallas_call(
        _pool_kernel,
        grid=(1,),
        in_specs=[
            full((N_PAD, H)),
            full((1, N_PAD)),
            full((H, H)),
            full((1, H)),
            full((H, 3)),
            full((1, 3)),
            full((B, 1)),
        ],
        out_specs=full((B, 1)),
        out_shape=jax.ShapeDtypeStruct((B, 1), jnp.float32),
    )(h, batch_f, ow1, ob1r, ow2p, ob2r, kind_f)


def kernel(protein_pos, protein_atom_feature, ligand_pos, ligand_atom_feature,
           batch_protein, batch_ligand, output_kind,
           Wp, bp, Wl, bl, eW1, eb1, eW2, eb2, infW, infb,
           nW1, nb1, nW2, nb2, oW1, ob1, oW2, ob2):
    npn = protein_pos.shape[0]
    nl = ligand_pos.shape[0]
    n = npn + nl
    dp = protein_atom_feature.shape[1]
    dl = ligand_atom_feature.shape[1]

    br = jnp.arange(B + 1, dtype=jnp.int32)
    sp = jnp.searchsorted(batch_protein, br, side="left").astype(jnp.int32)
    sl = jnp.searchsorted(batch_ligand, br, side="left").astype(jnp.int32)
    off = sp + sl

    q = jnp.arange(N_PAD, dtype=jnp.int32)
    valid = q < n
    bq = jnp.clip(jnp.searchsorted(off, q, side="right").astype(jnp.int32) - 1,
                  0, B - 1)
    npb = sp[bq + 1] - sp[bq]
    rq = q - off[bq]
    perm = jnp.where(rq < npb, sp[bq] + rq, npn + sl[bq] + (rq - npb))
    perm = jnp.where(valid, perm, 0)
    row_lo = jnp.where(valid, off[bq], 0).reshape(1, N_PAD)
    row_hi = jnp.where(valid, off[bq + 1], 0).reshape(1, N_PAD)
    batch_f = jnp.where(valid, bq, B).astype(jnp.float32).reshape(1, N_PAD)

    tstart = jnp.arange(T_KNN, dtype=jnp.int32) * R_KNN
    tlast = jnp.minimum(tstart + R_KNN - 1, n - 1)
    bfirst = bq[tstart]
    blast = bq[tlast]
    clo = (off[bfirst] // CW) * CW
    chi = ((off[blast + 1] + CW - 1) // CW) * CW
    nch = jnp.where(tstart < n, (chi - clo) // CW, 0)
    clo = jnp.where(tstart < n, clo, 0)

    zcol = lambda r, c: jnp.zeros((r, c), jnp.float32)
    prot = jnp.concatenate(
        [protein_atom_feature, zcol(npn, 40 - dp), jnp.ones((npn, 1)),
         zcol(npn, 1), zcol(npn, 6), protein_pos, zcol(npn, 77)], axis=1)
    lig = jnp.concatenate(
        [zcol(nl, dp), ligand_atom_feature, zcol(nl, 40 - dp - dl),
         zcol(nl, 1), jnp.ones((nl, 1)), zcol(nl, 6), ligand_pos,
         zcol(nl, 77)], axis=1)
    fpcomb = jnp.concatenate([prot, lig], axis=0)
    w128 = jnp.concatenate(
        [Wp, Wl, bp[None, :], bl[None, :], jnp.zeros((H - 42, H))],
        axis=0).astype(jnp.float32)

    sortedfp = _sc_gather(fpcomb, perm)
    posm = sortedfp[:, 48:56]
    pos_t = posm.T

    h = _embed(sortedfp, w128)

    idxkm, dkm = _knn(clo, nch, posm, pos_t, row_lo, row_hi)
    idx_flat = idxkm.reshape(K * N_PAD)
    d3 = dkm[:, :, None]

    for l in range(NUM_LAYERS):
        bv = _proj(h, eW1[l, NUM_G + H:, :])
        g = _sc_gather(bv, idx_flat)
        gr = g.reshape(K, N_PAD, H)
        h = _layer(
            h, d3, gr,
            eW1[l, :NUM_G, :], eW1[l, NUM_G:NUM_G + H, :],
            eb1[l].reshape(1, H), eW2[l], eb2[l].reshape(1, H),
            infW[l, :, 0].reshape(1, H),
            jnp.broadcast_to(infb[l].reshape(1, 1), (1, H)),
            nW1[l, :H, :], nW1[l, H:, :], nb1[l].reshape(1, H),
            nW2[l], nb2[l].reshape(1, H))

    return _pool(h, batch_f, oW1, ob1.reshape(1, H), oW2, ob2.reshape(1, 3),
                 output_kind.astype(jnp.float32).reshape(B, 1))


# speedup vs baseline: 8.5615x; 1.2834x over previous
import functools

import jax
import jax.numpy as jnp
from jax import lax
from jax.experimental import pallas as pl
from jax.experimental.pallas import tpu as pltpu
from jax.experimental.pallas import tpu_sc as plsc

NUM_LAYERS = 2
K = 32
NUM_G = 64
CUTOFF = 10.0
H = 128
B = 16

N_PAD = 10240
R_KNN = 128
CW = 256
T_KNN = N_PAD // R_KNN
R_E = 256
T_E = N_PAD // R_E

_GAP = CUTOFF / (NUM_G - 1)
_COEFF = -0.5 / (_GAP * _GAP)

_HIGH = jax.lax.Precision.HIGHEST


def _dot(a, b):
    return jnp.dot(a, b, precision=_HIGH, preferred_element_type=jnp.float32)


def _dot3(a, b):
    ah = a.astype(jnp.bfloat16)
    al = (a - ah.astype(jnp.float32)).astype(jnp.bfloat16)
    bh = b.astype(jnp.bfloat16)
    bl = (b - bh.astype(jnp.float32)).astype(jnp.bfloat16)
    f = jnp.float32
    d = lambda x, y: jnp.dot(x, y, preferred_element_type=f)
    return d(ah, bh) + (d(al, bh) + d(ah, bl))


def _sc_gather(values, flat_idx, window=256):
    num0 = flat_idx.shape[0]
    cols = values.shape[1]
    num = ((num0 + window * 32 - 1) // (window * 32)) * (window * 32)
    if num != num0:
        flat_idx = jnp.concatenate(
            [flat_idx, jnp.zeros((num - num0,), flat_idx.dtype)])
    idx2 = flat_idx.reshape(1, num)
    mesh = plsc.VectorSubcoreMesh(core_axis_name="c", subcore_axis_name="s")

    @pl.kernel(
        out_type=jax.ShapeDtypeStruct((num, cols), values.dtype),
        mesh=mesh,
    )
    def gk(x_hbm, i_hbm, o_hbm):
        def body(i_vmem, o_vmem):
            pltpu.sync_copy(x_hbm.at[i_vmem.at[0]], o_vmem)

        pltpu.emit_pipeline(
            body,
            grid=(num // window,),
            in_specs=[pl.BlockSpec((1, window), lambda i: (0, i))],
            out_specs=[pl.BlockSpec((window, cols), lambda i: (i, 0))],
            core_axis_name=("c", "s"),
            dimension_semantics=(pltpu.PARALLEL,),
        )(i_hbm, o_hbm)

    out = gk(values, idx2)
    return out[:num0] if num != num0 else out


def _embed_kernel(f_ref, w_ref, o_ref):
    o_ref[...] = _dot(f_ref[...], w_ref[...])


def _embed(feat, w48):
    return pl.pallas_call(
        _embed_kernel,
        grid=(T_E,),
        in_specs=[
            pl.BlockSpec((R_E, H), lambda t: (t, 0)),
            pl.BlockSpec((H, H), lambda t: (0, 0)),
        ],
        out_specs=pl.BlockSpec((R_E, H), lambda t: (t, 0)),
        out_shape=jax.ShapeDtypeStruct((N_PAD, H), jnp.float32),
    )(feat, w48)


def _proj(h, wb):
    return pl.pallas_call(
        _embed_kernel,
        grid=(T_E,),
        in_specs=[
            pl.BlockSpec((R_E, H), lambda t: (t, 0)),
            pl.BlockSpec((H, H), lambda t: (0, 0)),
        ],
        out_specs=pl.BlockSpec((R_E, H), lambda t: (t, 0)),
        out_shape=jax.ShapeDtypeStruct((N_PAD, H), jnp.float32),
    )(h, wb)


def _knn_kernel(clo_ref, nch_ref, posm_ref, post_ref, rlo_ref, rhi_ref,
                idx_ref, d_ref, buf_ref):
    t = pl.program_id(0)
    clo = clo_ref[t]
    nch = nch_ref[t]
    rows = t * R_KNN + lax.broadcasted_iota(jnp.int32, (1, R_KNN), 1)
    rlo = rlo_ref[...]
    rhi = rhi_ref[...]
    sub = lax.broadcasted_iota(jnp.int32, (CW, 1), 0)
    inf = jnp.float32(jnp.inf)
    ptile = post_ref[...]
    sqr = jnp.sum(ptile * ptile, axis=0, keepdims=True)

    def fill(j, _):
        base = pl.multiple_of(clo + j * CW, CW)
        cid = base + sub
        pc = posm_ref[pl.ds(base, CW), :]
        crs = jnp.dot(pc.astype(jnp.bfloat16), ptile.astype(jnp.bfloat16),
                      preferred_element_type=jnp.float32)
        sqc = jnp.sum(pc * pc, axis=1, keepdims=True)
        d2 = (sqc + sqr) - 2.0 * crs
        masked = (cid < rlo) | (cid >= rhi) | (cid == rows)
        buf_ref[pl.ds(base, CW), :] = jnp.where(masked, inf, d2)
        return 0

    lax.fori_loop(0, nch, fill, 0)

    prev = None
    for k in range(K):
        def chunk(j, carry, prev=prev):
            val, vidx = carry
            base = pl.multiple_of(clo + j * CW, CW)
            cid = base + sub
            c = buf_ref[pl.ds(base, CW), :]
            if prev is not None:
                c = jnp.where(cid == prev, inf, c)
                buf_ref[pl.ds(base, CW), :] = c
            mval = jnp.min(c, axis=0, keepdims=True)
            midx = jnp.min(jnp.where(c == mval, cid, N_PAD), axis=0,
                           keepdims=True)
            upd = mval < val
            return (jnp.where(upd, mval, val), jnp.where(upd, midx, vidx))

        val, vidx = lax.fori_loop(
            0, nch, chunk,
            (jnp.full((1, R_KNN), inf), jnp.zeros((1, R_KNN), jnp.int32)))
        idx_ref[k:k + 1, :] = vidx
        d_ref[k:k + 1, :] = jnp.sqrt(val)
        prev = vidx


def _knn(clo, nch, posm, pos_t, rlo, rhi):
    return pl.pallas_call(
        _knn_kernel,
        grid=(T_KNN,),
        in_specs=[
            pl.BlockSpec(memory_space=pltpu.SMEM),
            pl.BlockSpec(memory_space=pltpu.SMEM),
            pl.BlockSpec((N_PAD, 8), lambda t: (0, 0)),
            pl.BlockSpec((8, R_KNN), lambda t: (0, t)),
            pl.BlockSpec((1, R_KNN), lambda t: (0, t)),
            pl.BlockSpec((1, R_KNN), lambda t: (0, t)),
        ],
        out_specs=[
            pl.BlockSpec((K, R_KNN), lambda t: (0, t)),
            pl.BlockSpec((K, R_KNN), lambda t: (0, t)),
        ],
        out_shape=[
            jax.ShapeDtypeStruct((K, N_PAD), jnp.int32),
            jax.ShapeDtypeStruct((K, N_PAD), jnp.float32),
        ],
        scratch_shapes=[pltpu.VMEM((N_PAD, R_KNN), jnp.float32)],
    )(clo, nch, posm, pos_t, rlo, rhi)


def _layer_kernel(h_ref, d_ref, g_ref, wg_ref, wa_ref, eb1_ref, ew2_ref,
                  eb2_ref, infw_ref, infb_ref, nw1a_ref, nw1b_ref, nb1_ref,
                  nw2_ref, nb2_ref, o_ref):
    h = h_ref[...]
    a = _dot(h, wa_ref[...]) + eb1_ref[...]
    offs = lax.broadcasted_iota(jnp.int32, (1, NUM_G), 1).astype(
        jnp.float32) * jnp.float32(_GAP)
    eb2 = eb2_ref[...]
    infw = infw_ref[...]
    infb = infb_ref[0:1, 0:1]
    wg = wg_ref[...]
    ew2 = ew2_ref[...]

    def body(k, mi):
        bk = g_ref[pl.ds(k, 1), :, :].reshape(R_E, H)
        dk = d_ref[pl.ds(k, 1), :, :].reshape(R_E, 1)
        rbf = jnp.exp(jnp.float32(_COEFF) * (dk - offs) ** 2)
        m1 = jnp.maximum(_dot3(rbf, wg) + a + bk, 0.0)
        m2 = jnp.maximum(_dot3(m1, ew2) + eb2, 0.0)
        s = jnp.sum(m2 * infw, axis=-1, keepdims=True)
        gate = jax.nn.sigmoid(s + infb)
        return mi + gate * m2

    mi = lax.fori_loop(0, K, body, jnp.zeros((R_E, H), jnp.float32))
    n1 = jnp.maximum(_dot(mi, nw1a_ref[...]) + _dot(h, nw1b_ref[...])
                     + nb1_ref[...], 0.0)
    o_ref[...] = h + _dot(n1, nw2_ref[...]) + nb2_ref[...]


def _layer(h, dkm, gr, wg, wa, eb1r, ew2, eb2r, infwr, infbr, nw1a, nw1b,
           nb1r, nw2, nb2r):
    npart = h.shape[0]
    full = lambda shape: pl.BlockSpec(shape, lambda t: tuple(0 for _ in shape))
    return pl.pallas_call(
        _layer_kernel,
        grid=(npart // R_E,),
        in_specs=[
            pl.BlockSpec((R_E, H), lambda t: (t, 0)),
            pl.BlockSpec((K, R_E, 1), lambda t: (0, t, 0)),
            pl.BlockSpec((K, R_E, H), lambda t: (0, t, 0)),
            full((NUM_G, H)),
            full((H, H)),
            full((1, H)),
            full((H, H)),
            full((1, H)),
            full((1, H)),
            full((1, H)),
            full((H, H)),
            full((H, H)),
            full((1, H)),
            full((H, H)),
            full((1, H)),
        ],
        out_specs=pl.BlockSpec((R_E, H), lambda t: (t, 0)),
        out_shape=jax.ShapeDtypeStruct((npart, H), jnp.float32),
    )(h, dkm, gr, wg, wa, eb1r, ew2, eb2r, infwr, infbr, nw1a, nw1b, nb1r,
      nw2, nb2r)


def _pool_kernel(h_ref, bt_ref, ow1_ref, ob1_ref, ow2_ref, ob2_ref, kind_ref,
                 o_ref):
    bt = bt_ref[...]
    gid = lax.broadcasted_iota(jnp.int32, (B, 1), 0).astype(jnp.float32)
    onehot = jnp.where(bt == gid, 1.0, 0.0)
    pre = _dot(onehot, h_ref[...])
    o = _dot(pre, ow1_ref[...]) + ob1_ref[...]
    o = jax.nn.softplus(o) - jnp.float32(jnp.log(2.0))
    o = _dot(o, ow2_ref[...]) + ob2_ref[...]
    kidx = lax.broadcasted_iota(jnp.int32, (B, 3), 1).astype(jnp.float32)
    mask = jnp.where(kidx == kind_ref[...] - 1.0, 1.0, 0.0)
    o_ref[...] = jnp.sum(o * mask, axis=-1, keepdims=True)


def _pool(h, batch_f, ow1, ob1r, ow2p, ob2r, kind_f):
    full = lambda shape: pl.BlockSpec(shape, lambda t: tuple(0 for _ in shape))
    return pl.pallas_call(
        _pool_kernel,
        grid=(1,),
        in_specs=[
            full((N_PAD, H)),
            full((1, N_PAD)),
            full((H, H)),
            full((1, H)),
            full((H, 3)),
            full((1, 3)),
            full((B, 1)),
        ],
        out_specs=full((B, 1)),
        out_shape=jax.ShapeDtypeStruct((B, 1), jnp.float32),
    )(h, batch_f, ow1, ob1r, ow2p, ob2r, kind_f)


def kernel(protein_pos, protein_atom_feature, ligand_pos, ligand_atom_feature,
           batch_protein, batch_ligand, output_kind,
           Wp, bp, Wl, bl, eW1, eb1, eW2, eb2, infW, infb,
           nW1, nb1, nW2, nb2, oW1, ob1, oW2, ob2):
    npn = protein_pos.shape[0]
    nl = ligand_pos.shape[0]
    n = npn + nl
    dp = protein_atom_feature.shape[1]
    dl = ligand_atom_feature.shape[1]

    br = jnp.arange(B + 1, dtype=jnp.int32)
    sp = jnp.searchsorted(batch_protein, br, side="left").astype(jnp.int32)
    sl = jnp.searchsorted(batch_ligand, br, side="left").astype(jnp.int32)
    off = sp + sl

    q = jnp.arange(N_PAD, dtype=jnp.int32)
    valid = q < n
    bq = jnp.clip(jnp.searchsorted(off, q, side="right").astype(jnp.int32) - 1,
                  0, B - 1)
    npb = sp[bq + 1] - sp[bq]
    rq = q - off[bq]
    perm = jnp.where(rq < npb, sp[bq] + rq, npn + sl[bq] + (rq - npb))
    perm = jnp.where(valid, perm, 0)
    row_lo = jnp.where(valid, off[bq], 0).reshape(1, N_PAD)
    row_hi = jnp.where(valid, off[bq + 1], 0).reshape(1, N_PAD)
    batch_f = jnp.where(valid, bq, B).astype(jnp.float32).reshape(1, N_PAD)

    tstart = jnp.arange(T_KNN, dtype=jnp.int32) * R_KNN
    tlast = jnp.minimum(tstart + R_KNN - 1, n - 1)
    bfirst = bq[tstart]
    blast = bq[tlast]
    clo = (off[bfirst] // CW) * CW
    chi = ((off[blast + 1] + CW - 1) // CW) * CW
    nch = jnp.where(tstart < n, (chi - clo) // CW, 0)
    clo = jnp.where(tstart < n, clo, 0)

    zcol = lambda r, c: jnp.zeros((r, c), jnp.float32)
    prot = jnp.concatenate(
        [protein_atom_feature, zcol(npn, 40 - dp), jnp.ones((npn, 1)),
         zcol(npn, 1), zcol(npn, 6), protein_pos, zcol(npn, 77)], axis=1)
    lig = jnp.concatenate(
        [zcol(nl, dp), ligand_atom_feature, zcol(nl, 40 - dp - dl),
         zcol(nl, 1), jnp.ones((nl, 1)), zcol(nl, 6), ligand_pos,
         zcol(nl, 77)], axis=1)
    fpcomb = jnp.concatenate([prot, lig], axis=0)
    w128 = jnp.concatenate(
        [Wp, Wl, bp[None, :], bl[None, :], jnp.zeros((H - 42, H))],
        axis=0).astype(jnp.float32)

    sortedfp = _sc_gather(fpcomb, perm, window=128)
    posm = sortedfp[:, 48:56]
    pos_t = posm.T

    h = _embed(sortedfp, w128)

    idxkm, dkm = _knn(clo, nch, posm, pos_t, row_lo, row_hi)
    idx_flat = idxkm.reshape(K * N_PAD)
    d3 = dkm[:, :, None]

    half = N_PAD // 2
    idx_a = idxkm[:, :half].reshape(K * half)
    idx_b = idxkm[:, half:].reshape(K * half)
    d3_a, d3_b = d3[:, :half], d3[:, half:]

    for l in range(NUM_LAYERS):
        wargs = (
            eW1[l, :NUM_G, :], eW1[l, NUM_G:NUM_G + H, :],
            eb1[l].reshape(1, H), eW2[l], eb2[l].reshape(1, H),
            infW[l, :, 0].reshape(1, H),
            jnp.broadcast_to(infb[l].reshape(1, 1), (1, H)),
            nW1[l, :H, :], nW1[l, H:, :], nb1[l].reshape(1, H),
            nW2[l], nb2[l].reshape(1, H))
        bv = _proj(h, eW1[l, NUM_G + H:, :])
        ga = _sc_gather(bv, idx_a).reshape(K, half, H)
        gb = _sc_gather(bv, idx_b).reshape(K, half, H)
        ha = _layer(h[:half], d3_a, ga, *wargs)
        hb = _layer(h[half:], d3_b, gb, *wargs)
        h = jnp.concatenate([ha, hb], axis=0)

    return _pool(h, batch_f, oW1, ob1.reshape(1, H), oW2, ob2.reshape(1, 3),
                 output_kind.astype(jnp.float32).reshape(B, 1))


# speedup vs baseline: 8.7608x; 1.0233x over previous
import functools

import jax
import jax.numpy as jnp
from jax import lax
from jax.experimental import pallas as pl
from jax.experimental.pallas import tpu as pltpu
from jax.experimental.pallas import tpu_sc as plsc

NUM_LAYERS = 2
K = 32
NUM_G = 64
CUTOFF = 10.0
H = 128
B = 16

N_PAD = 10240
R_KNN = 256
CW = 256
T_KNN = N_PAD // R_KNN
R_E = 256
T_E = N_PAD // R_E

_GAP = CUTOFF / (NUM_G - 1)
_COEFF = -0.5 / (_GAP * _GAP)

_HIGH = jax.lax.Precision.HIGHEST


def _dot(a, b):
    return jnp.dot(a, b, precision=_HIGH, preferred_element_type=jnp.float32)


def _dot3(a, b):
    ah = a.astype(jnp.bfloat16)
    al = (a - ah.astype(jnp.float32)).astype(jnp.bfloat16)
    bh = b.astype(jnp.bfloat16)
    bl = (b - bh.astype(jnp.float32)).astype(jnp.bfloat16)
    f = jnp.float32
    d = lambda x, y: jnp.dot(x, y, preferred_element_type=f)
    return d(ah, bh) + (d(al, bh) + d(ah, bl))


def _sc_gather(values, flat_idx, window=256):
    num0 = flat_idx.shape[0]
    cols = values.shape[1]
    num = ((num0 + window * 32 - 1) // (window * 32)) * (window * 32)
    if num != num0:
        flat_idx = jnp.concatenate(
            [flat_idx, jnp.zeros((num - num0,), flat_idx.dtype)])
    idx2 = flat_idx.reshape(1, num)
    mesh = plsc.VectorSubcoreMesh(core_axis_name="c", subcore_axis_name="s")

    @pl.kernel(
        out_type=jax.ShapeDtypeStruct((num, cols), values.dtype),
        mesh=mesh,
    )
    def gk(x_hbm, i_hbm, o_hbm):
        def body(i_vmem, o_vmem):
            pltpu.sync_copy(x_hbm.at[i_vmem.at[0]], o_vmem)

        pltpu.emit_pipeline(
            body,
            grid=(num // window,),
            in_specs=[pl.BlockSpec((1, window), lambda i: (0, i))],
            out_specs=[pl.BlockSpec((window, cols), lambda i: (i, 0))],
            core_axis_name=("c", "s"),
            dimension_semantics=(pltpu.PARALLEL,),
        )(i_hbm, o_hbm)

    out = gk(values, idx2)
    return out[:num0] if num != num0 else out


def _embed_kernel(f_ref, w_ref, o_ref):
    o_ref[...] = _dot(f_ref[...], w_ref[...])


def _embed(feat, w48):
    return pl.pallas_call(
        _embed_kernel,
        grid=(T_E,),
        in_specs=[
            pl.BlockSpec((R_E, H), lambda t: (t, 0)),
            pl.BlockSpec((H, H), lambda t: (0, 0)),
        ],
        out_specs=pl.BlockSpec((R_E, H), lambda t: (t, 0)),
        out_shape=jax.ShapeDtypeStruct((N_PAD, H), jnp.float32),
    )(feat, w48)


def _proj(h, wb):
    return pl.pallas_call(
        _embed_kernel,
        grid=(T_E,),
        in_specs=[
            pl.BlockSpec((R_E, H), lambda t: (t, 0)),
            pl.BlockSpec((H, H), lambda t: (0, 0)),
        ],
        out_specs=pl.BlockSpec((R_E, H), lambda t: (t, 0)),
        out_shape=jax.ShapeDtypeStruct((N_PAD, H), jnp.float32),
    )(h, wb)


def _knn_kernel(clo_ref, nch_ref, posm_ref, post_ref, rlo_ref, rhi_ref,
                idx_ref, d_ref, buf_ref):
    t = pl.program_id(0)
    clo = clo_ref[t]
    nch = nch_ref[t]
    rows = t * R_KNN + lax.broadcasted_iota(jnp.int32, (1, R_KNN), 1)
    rlo = rlo_ref[...]
    rhi = rhi_ref[...]
    sub = lax.broadcasted_iota(jnp.int32, (CW, 1), 0)
    inf = jnp.float32(jnp.inf)
    ptile = post_ref[...]
    sqr = jnp.sum(ptile * ptile, axis=0, keepdims=True)

    def fill(j, _):
        base = pl.multiple_of(clo + j * CW, CW)
        cid = base + sub
        pc = posm_ref[pl.ds(base, CW), :]
        crs = jnp.dot(pc.astype(jnp.bfloat16), ptile.astype(jnp.bfloat16),
                      preferred_element_type=jnp.float32)
        sqc = jnp.sum(pc * pc, axis=1, keepdims=True)
        d2 = (sqc + sqr) - 2.0 * crs
        masked = (cid < rlo) | (cid >= rhi) | (cid == rows)
        buf_ref[pl.ds(base, CW), :] = jnp.where(masked, inf, d2)
        return 0

    lax.fori_loop(0, nch, fill, 0)

    prev = None
    for k in range(K):
        def chunk(j, carry, prev=prev):
            val, vidx = carry
            base = pl.multiple_of(clo + j * CW, CW)
            cid = base + sub
            c = buf_ref[pl.ds(base, CW), :]
            if prev is not None:
                c = jnp.where(cid == prev, inf, c)
                buf_ref[pl.ds(base, CW), :] = c
            mval = jnp.min(c, axis=0, keepdims=True)
            midx = jnp.min(jnp.where(c == mval, cid, N_PAD), axis=0,
                           keepdims=True)
            upd = mval < val
            return (jnp.where(upd, mval, val), jnp.where(upd, midx, vidx))

        val, vidx = lax.fori_loop(
            0, nch, chunk,
            (jnp.full((1, R_KNN), inf), jnp.zeros((1, R_KNN), jnp.int32)))
        idx_ref[k:k + 1, :] = vidx
        d_ref[k:k + 1, :] = jnp.sqrt(val)
        prev = vidx


def _knn(clo, nch, posm, pos_t, rlo, rhi):
    return pl.pallas_call(
        _knn_kernel,
        grid=(T_KNN,),
        in_specs=[
            pl.BlockSpec(memory_space=pltpu.SMEM),
            pl.BlockSpec(memory_space=pltpu.SMEM),
            pl.BlockSpec((N_PAD, 8), lambda t: (0, 0)),
            pl.BlockSpec((8, R_KNN), lambda t: (0, t)),
            pl.BlockSpec((1, R_KNN), lambda t: (0, t)),
            pl.BlockSpec((1, R_KNN), lambda t: (0, t)),
        ],
        out_specs=[
            pl.BlockSpec((K, R_KNN), lambda t: (0, t)),
            pl.BlockSpec((K, R_KNN), lambda t: (0, t)),
        ],
        out_shape=[
            jax.ShapeDtypeStruct((K, N_PAD), jnp.int32),
            jax.ShapeDtypeStruct((K, N_PAD), jnp.float32),
        ],
        scratch_shapes=[pltpu.VMEM((N_PAD, R_KNN), jnp.float32)],
    )(clo, nch, posm, pos_t, rlo, rhi)


def _layer_kernel(h_ref, d_ref, g_ref, wg_ref, wa_ref, eb1_ref, ew2_ref,
                  eb2_ref, infw_ref, infb_ref, nw1a_ref, nw1b_ref, nb1_ref,
                  nw2_ref, nb2_ref, o_ref):
    h = h_ref[...]
    a = _dot(h, wa_ref[...]) + eb1_ref[...]
    offs = lax.broadcasted_iota(jnp.int32, (1, NUM_G), 1).astype(
        jnp.float32) * jnp.float32(_GAP)
    eb2 = eb2_ref[...]
    infw = infw_ref[...]
    infb = infb_ref[0:1, 0:1]
    wg = wg_ref[...]
    ew2 = ew2_ref[...]

    def body(k, mi):
        bk = g_ref[pl.ds(k, 1), :, :].reshape(R_E, H)
        dk = d_ref[pl.ds(k, 1), :, :].reshape(R_E, 1)
        rbf = jnp.exp(jnp.float32(_COEFF) * (dk - offs) ** 2)
        m1 = jnp.maximum(_dot3(rbf, wg) + a + bk, 0.0)
        m2 = jnp.maximum(_dot3(m1, ew2) + eb2, 0.0)
        s = jnp.sum(m2 * infw, axis=-1, keepdims=True)
        gate = jax.nn.sigmoid(s + infb)
        return mi + gate * m2

    mi = lax.fori_loop(0, K, body, jnp.zeros((R_E, H), jnp.float32))
    n1 = jnp.maximum(_dot(mi, nw1a_ref[...]) + _dot(h, nw1b_ref[...])
                     + nb1_ref[...], 0.0)
    o_ref[...] = h + _dot(n1, nw2_ref[...]) + nb2_ref[...]


def _layer(h, dkm, gr, wg, wa, eb1r, ew2, eb2r, infwr, infbr, nw1a, nw1b,
           nb1r, nw2, nb2r):
    npart = h.shape[0]
    full = lambda shape: pl.BlockSpec(shape, lambda t: tuple(0 for _ in shape))
    return pl.pallas_call(
        _layer_kernel,
        grid=(npart // R_E,),
        in_specs=[
            pl.BlockSpec((R_E, H), lambda t: (t, 0)),
            pl.BlockSpec((K, R_E, 1), lambda t: (0, t, 0)),
            pl.BlockSpec((K, R_E, H), lambda t: (0, t, 0)),
            full((NUM_G, H)),
            full((H, H)),
            full((1, H)),
            full((H, H)),
            full((1, H)),
            full((1, H)),
            full((1, H)),
            full((H, H)),
            full((H, H)),
            full((1, H)),
            full((H, H)),
            full((1, H)),
        ],
        out_specs=pl.BlockSpec((R_E, H), lambda t: (t, 0)),
        out_shape=jax.ShapeDtypeStruct((npart, H), jnp.float32),
    )(h, dkm, gr, wg, wa, eb1r, ew2, eb2r, infwr, infbr, nw1a, nw1b, nb1r,
      nw2, nb2r)


def _pool_kernel(h_ref, bt_ref, ow1_ref, ob1_ref, ow2_ref, ob2_ref, kind_ref,
                 o_ref):
    bt = bt_ref[...]
    gid = lax.broadcasted_iota(jnp.int32, (B, 1), 0).astype(jnp.float32)
    onehot = jnp.where(bt == gid, 1.0, 0.0)
    pre = _dot(onehot, h_ref[...])
    o = _dot(pre, ow1_ref[...]) + ob1_ref[...]
    o = jax.nn.softplus(o) - jnp.float32(jnp.log(2.0))
    o = _dot(o, ow2_ref[...]) + ob2_ref[...]
    kidx = lax.broadcasted_iota(jnp.int32, (B, 3), 1).astype(jnp.float32)
    mask = jnp.where(kidx == kind_ref[...] - 1.0, 1.0, 0.0)
    o_ref[...] = jnp.sum(o * mask, axis=-1, keepdims=True)


def _pool(h, batch_f, ow1, ob1r, ow2p, ob2r, kind_f):
    full = lambda shape: pl.BlockSpec(shape, lambda t: tuple(0 for _ in shape))
    return pl.pallas_call(
        _pool_kernel,
        grid=(1,),
        in_specs=[
            full((N_PAD, H)),
            full((1, N_PAD)),
            full((H, H)),
            full((1, H)),
            full((H, 3)),
            full((1, 3)),
            full((B, 1)),
        ],
        out_specs=full((B, 1)),
        out_shape=jax.ShapeDtypeStruct((B, 1), jnp.float32),
    )(h, batch_f, ow1, ob1r, ow2p, ob2r, kind_f)


def kernel(protein_pos, protein_atom_feature, ligand_pos, ligand_atom_feature,
           batch_protein, batch_ligand, output_kind,
           Wp, bp, Wl, bl, eW1, eb1, eW2, eb2, infW, infb,
           nW1, nb1, nW2, nb2, oW1, ob1, oW2, ob2):
    npn = protein_pos.shape[0]
    nl = ligand_pos.shape[0]
    n = npn + nl
    dp = protein_atom_feature.shape[1]
    dl = ligand_atom_feature.shape[1]

    br = jnp.arange(B + 1, dtype=jnp.int32)
    sp = jnp.searchsorted(batch_protein, br, side="left").astype(jnp.int32)
    sl = jnp.searchsorted(batch_ligand, br, side="left").astype(jnp.int32)
    off = sp + sl

    q = jnp.arange(N_PAD, dtype=jnp.int32)
    valid = q < n
    bq = jnp.clip(jnp.searchsorted(off, q, side="right").astype(jnp.int32) - 1,
                  0, B - 1)
    npb = sp[bq + 1] - sp[bq]
    rq = q - off[bq]
    perm = jnp.where(rq < npb, sp[bq] + rq, npn + sl[bq] + (rq - npb))
    perm = jnp.where(valid, perm, 0)
    row_lo = jnp.where(valid, off[bq], 0).reshape(1, N_PAD)
    row_hi = jnp.where(valid, off[bq + 1], 0).reshape(1, N_PAD)
    batch_f = jnp.where(valid, bq, B).astype(jnp.float32).reshape(1, N_PAD)

    tstart = jnp.arange(T_KNN, dtype=jnp.int32) * R_KNN
    tlast = jnp.minimum(tstart + R_KNN - 1, n - 1)
    bfirst = bq[tstart]
    blast = bq[tlast]
    clo = (off[bfirst] // CW) * CW
    chi = ((off[blast + 1] + CW - 1) // CW) * CW
    nch = jnp.where(tstart < n, (chi - clo) // CW, 0)
    clo = jnp.where(tstart < n, clo, 0)

    zcol = lambda r, c: jnp.zeros((r, c), jnp.float32)
    prot = jnp.concatenate(
        [protein_atom_feature, zcol(npn, 40 - dp), jnp.ones((npn, 1)),
         zcol(npn, 1), zcol(npn, 6), protein_pos, zcol(npn, 77)], axis=1)
    lig = jnp.concatenate(
        [zcol(nl, dp), ligand_atom_feature, zcol(nl, 40 - dp - dl),
         zcol(nl, 1), jnp.ones((nl, 1)), zcol(nl, 6), ligand_pos,
         zcol(nl, 77)], axis=1)
    fpcomb = jnp.concatenate([prot, lig], axis=0)
    w128 = jnp.concatenate(
        [Wp, Wl, bp[None, :], bl[None, :], jnp.zeros((H - 42, H))],
        axis=0).astype(jnp.float32)

    sortedfp = _sc_gather(fpcomb, perm, window=128)
    posm = sortedfp[:, 48:56]
    pos_t = posm.T

    h = _embed(sortedfp, w128)

    idxkm, dkm = _knn(clo, nch, posm, pos_t, row_lo, row_hi)
    idx_flat = idxkm.reshape(K * N_PAD)
    d3 = dkm[:, :, None]

    half = N_PAD // 2
    idx_a = idxkm[:, :half].reshape(K * half)
    idx_b = idxkm[:, half:].reshape(K * half)
    d3_a, d3_b = d3[:, :half], d3[:, half:]

    for l in range(NUM_LAYERS):
        wargs = (
            eW1[l, :NUM_G, :], eW1[l, NUM_G:NUM_G + H, :],
            eb1[l].reshape(1, H), eW2[l], eb2[l].reshape(1, H),
            infW[l, :, 0].reshape(1, H),
            jnp.broadcast_to(infb[l].reshape(1, 1), (1, H)),
            nW1[l, :H, :], nW1[l, H:, :], nb1[l].reshape(1, H),
            nW2[l], nb2[l].reshape(1, H))
        bv = _proj(h, eW1[l, NUM_G + H:, :])
        ga = _sc_gather(bv, idx_a).reshape(K, half, H)
        gb = _sc_gather(bv, idx_b).reshape(K, half, H)
        ha = _layer(h[:half], d3_a, ga, *wargs)
        hb = _layer(h[half:], d3_b, gb, *wargs)
        h = jnp.concatenate([ha, hb], axis=0)

    return _pool(h, batch_f, oW1, ob1.reshape(1, H), oW2, ob2.reshape(1, 3),
                 output_kind.astype(jnp.float32).reshape(B, 1))


# speedup vs baseline: 8.7638x; 1.0003x over previous
import functools

import jax
import jax.numpy as jnp
from jax import lax
from jax.experimental import pallas as pl
from jax.experimental.pallas import tpu as pltpu
from jax.experimental.pallas import tpu_sc as plsc

NUM_LAYERS = 2
K = 32
NUM_G = 64
CUTOFF = 10.0
H = 128
B = 16

N_PAD = 10240
R_KNN = 256
CW = 256
T_KNN = N_PAD // R_KNN
R_E = 256
T_E = N_PAD // R_E

_GAP = CUTOFF / (NUM_G - 1)
_COEFF = -0.5 / (_GAP * _GAP)

_HIGH = jax.lax.Precision.HIGHEST


def _dot(a, b):
    return jnp.dot(a, b, precision=_HIGH, preferred_element_type=jnp.float32)


def _split(a):
    ah = a.astype(jnp.bfloat16)
    al = (a - ah.astype(jnp.float32)).astype(jnp.bfloat16)
    return ah, al


def _dot3p(a, bh, bl):
    ah, al = _split(a)
    d = lambda x, y: jnp.dot(x, y, preferred_element_type=jnp.float32)
    return d(ah, bh) + (d(al, bh) + d(ah, bl))


def _dot3(a, b):
    bh, bl = _split(b)
    return _dot3p(a, bh, bl)


def _sc_gather(values, flat_idx, window=256):
    num0 = flat_idx.shape[0]
    cols = values.shape[1]
    num = ((num0 + window * 32 - 1) // (window * 32)) * (window * 32)
    if num != num0:
        flat_idx = jnp.concatenate(
            [flat_idx, jnp.zeros((num - num0,), flat_idx.dtype)])
    idx2 = flat_idx.reshape(1, num)
    mesh = plsc.VectorSubcoreMesh(core_axis_name="c", subcore_axis_name="s")

    @pl.kernel(
        out_type=jax.ShapeDtypeStruct((num, cols), values.dtype),
        mesh=mesh,
    )
    def gk(x_hbm, i_hbm, o_hbm):
        def body(i_vmem, o_vmem):
            pltpu.sync_copy(x_hbm.at[i_vmem.at[0]], o_vmem)

        pltpu.emit_pipeline(
            body,
            grid=(num // window,),
            in_specs=[pl.BlockSpec((1, window), lambda i: (0, i))],
            out_specs=[pl.BlockSpec((window, cols), lambda i: (i, 0))],
            core_axis_name=("c", "s"),
            dimension_semantics=(pltpu.PARALLEL,),
        )(i_hbm, o_hbm)

    out = gk(values, idx2)
    return out[:num0] if num != num0 else out


def _embed_kernel(f_ref, w_ref, o_ref):
    o_ref[...] = _dot(f_ref[...], w_ref[...])


def _embed(feat, w48):
    return pl.pallas_call(
        _embed_kernel,
        grid=(T_E,),
        in_specs=[
            pl.BlockSpec((R_E, H), lambda t: (t, 0)),
            pl.BlockSpec((H, H), lambda t: (0, 0)),
        ],
        out_specs=pl.BlockSpec((R_E, H), lambda t: (t, 0)),
        out_shape=jax.ShapeDtypeStruct((N_PAD, H), jnp.float32),
    )(feat, w48)


def _proj(h, wb):
    return pl.pallas_call(
        _embed_kernel,
        grid=(T_E,),
        in_specs=[
            pl.BlockSpec((R_E, H), lambda t: (t, 0)),
            pl.BlockSpec((H, H), lambda t: (0, 0)),
        ],
        out_specs=pl.BlockSpec((R_E, H), lambda t: (t, 0)),
        out_shape=jax.ShapeDtypeStruct((N_PAD, H), jnp.float32),
    )(h, wb)


def _knn_kernel(clo_ref, nch_ref, posm_ref, post_ref, rlo_ref, rhi_ref,
                idx_ref, d_ref, buf_ref):
    t = pl.program_id(0)
    clo = clo_ref[t]
    nch = nch_ref[t]
    rows = t * R_KNN + lax.broadcasted_iota(jnp.int32, (1, R_KNN), 1)
    rlo = rlo_ref[...]
    rhi = rhi_ref[...]
    sub = lax.broadcasted_iota(jnp.int32, (CW, 1), 0)
    inf = jnp.float32(jnp.inf)
    ptile = post_ref[...]
    sqr = jnp.sum(ptile * ptile, axis=0, keepdims=True)

    def fill(j, _):
        base = pl.multiple_of(clo + j * CW, CW)
        cid = base + sub
        pc = posm_ref[pl.ds(base, CW), :]
        crs = jnp.dot(pc.astype(jnp.bfloat16), ptile.astype(jnp.bfloat16),
                      preferred_element_type=jnp.float32)
        sqc = jnp.sum(pc * pc, axis=1, keepdims=True)
        d2 = (sqc + sqr) - 2.0 * crs
        masked = (cid < rlo) | (cid >= rhi) | (cid == rows)
        buf_ref[pl.ds(base, CW), :] = jnp.where(masked, inf, d2)
        return 0

    lax.fori_loop(0, nch, fill, 0)

    prev = None
    for k in range(K):
        def chunk(j, carry, prev=prev):
            val, vidx = carry
            base = pl.multiple_of(clo + j * CW, CW)
            cid = base + sub
            c = buf_ref[pl.ds(base, CW), :]
            if prev is not None:
                c = jnp.where(cid == prev, inf, c)
                buf_ref[pl.ds(base, CW), :] = c
            mval = jnp.min(c, axis=0, keepdims=True)
            midx = jnp.min(jnp.where(c == mval, cid, N_PAD), axis=0,
                           keepdims=True)
            upd = mval < val
            return (jnp.where(upd, mval, val), jnp.where(upd, midx, vidx))

        val, vidx = lax.fori_loop(
            0, nch, chunk,
            (jnp.full((1, R_KNN), inf), jnp.zeros((1, R_KNN), jnp.int32)))
        idx_ref[k:k + 1, :] = vidx
        d_ref[k:k + 1, :] = jnp.sqrt(val)
        prev = vidx


def _knn(clo, nch, posm, pos_t, rlo, rhi):
    return pl.pallas_call(
        _knn_kernel,
        grid=(T_KNN,),
        in_specs=[
            pl.BlockSpec(memory_space=pltpu.SMEM),
            pl.BlockSpec(memory_space=pltpu.SMEM),
            pl.BlockSpec((N_PAD, 8), lambda t: (0, 0)),
            pl.BlockSpec((8, R_KNN), lambda t: (0, t)),
            pl.BlockSpec((1, R_KNN), lambda t: (0, t)),
            pl.BlockSpec((1, R_KNN), lambda t: (0, t)),
        ],
        out_specs=[
            pl.BlockSpec((K, R_KNN), lambda t: (0, t)),
            pl.BlockSpec((K, R_KNN), lambda t: (0, t)),
        ],
        out_shape=[
            jax.ShapeDtypeStruct((K, N_PAD), jnp.int32),
            jax.ShapeDtypeStruct((K, N_PAD), jnp.float32),
        ],
        scratch_shapes=[pltpu.VMEM((N_PAD, R_KNN), jnp.float32)],
    )(clo, nch, posm, pos_t, rlo, rhi)


def _layer_kernel(h_ref, d_ref, g_ref, wg_ref, wa_ref, eb1_ref, ew2_ref,
                  eb2_ref, infw_ref, infb_ref, nw1a_ref, nw1b_ref, nb1_ref,
                  nw2_ref, nb2_ref, o_ref):
    h = h_ref[...]
    a = _dot(h, wa_ref[...]) + eb1_ref[...]
    offs = lax.broadcasted_iota(jnp.int32, (1, NUM_G), 1).astype(
        jnp.float32) * jnp.float32(_GAP)
    eb2 = eb2_ref[...]
    infw = infw_ref[...]
    infb = infb_ref[0:1, 0:1]
    wgh, wgl = _split(wg_ref[...])
    ew2h, ew2l = _split(ew2_ref[...])

    def body(k, mi):
        bk = g_ref[pl.ds(k, 1), :, :].reshape(R_E, H)
        dk = d_ref[pl.ds(k, 1), :, :].reshape(R_E, 1)
        rbf = jnp.exp(jnp.float32(_COEFF) * (dk - offs) ** 2)
        m1 = jnp.maximum(_dot3p(rbf, wgh, wgl) + a + bk, 0.0)
        m2 = jnp.maximum(_dot3p(m1, ew2h, ew2l) + eb2, 0.0)
        s = jnp.sum(m2 * infw, axis=-1, keepdims=True)
        gate = jax.nn.sigmoid(s + infb)
        return mi + gate * m2

    mi = lax.fori_loop(0, K, body, jnp.zeros((R_E, H), jnp.float32))
    n1 = jnp.maximum(_dot(mi, nw1a_ref[...]) + _dot(h, nw1b_ref[...])
                     + nb1_ref[...], 0.0)
    o_ref[...] = h + _dot(n1, nw2_ref[...]) + nb2_ref[...]


def _layer(h, dkm, gr, wg, wa, eb1r, ew2, eb2r, infwr, infbr, nw1a, nw1b,
           nb1r, nw2, nb2r):
    npart = h.shape[0]
    full = lambda shape: pl.BlockSpec(shape, lambda t: tuple(0 for _ in shape))
    return pl.pallas_call(
        _layer_kernel,
        grid=(npart // R_E,),
        in_specs=[
            pl.BlockSpec((R_E, H), lambda t: (t, 0)),
            pl.BlockSpec((K, R_E, 1), lambda t: (0, t, 0)),
            pl.BlockSpec((K, R_E, H), lambda t: (0, t, 0)),
            full((NUM_G, H)),
            full((H, H)),
            full((1, H)),
            full((H, H)),
            full((1, H)),
            full((1, H)),
            full((1, H)),
            full((H, H)),
            full((H, H)),
            full((1, H)),
            full((H, H)),
            full((1, H)),
        ],
        out_specs=pl.BlockSpec((R_E, H), lambda t: (t, 0)),
        out_shape=jax.ShapeDtypeStruct((npart, H), jnp.float32),
    )(h, dkm, gr, wg, wa, eb1r, ew2, eb2r, infwr, infbr, nw1a, nw1b, nb1r,
      nw2, nb2r)


def _pool_kernel(h_ref, bt_ref, ow1_ref, ob1_ref, ow2_ref, ob2_ref, kind_ref,
                 o_ref):
    bt = bt_ref[...]
    gid = lax.broadcasted_iota(jnp.int32, (B, 1), 0).astype(jnp.float32)
    onehot = jnp.where(bt == gid, 1.0, 0.0)
    pre = _dot(onehot, h_ref[...])
    o = _dot(pre, ow1_ref[...]) + ob1_ref[...]
    o = jax.nn.softplus(o) - jnp.float32(jnp.log(2.0))
    o = _dot(o, ow2_ref[...]) + ob2_ref[...]
    kidx = lax.broadcasted_iota(jnp.int32, (B, 3), 1).astype(jnp.float32)
    mask = jnp.where(kidx == kind_ref[...] - 1.0, 1.0, 0.0)
    o_ref[...] = jnp.sum(o * mask, axis=-1, keepdims=True)


def _pool(h, batch_f, ow1, ob1r, ow2p, ob2r, kind_f):
    full = lambda shape: pl.BlockSpec(shape, lambda t: tuple(0 for _ in shape))
    return pl.pallas_call(
        _pool_kernel,
        grid=(1,),
        in_specs=[
            full((N_PAD, H)),
            full((1, N_PAD)),
            full((H, H)),
            full((1, H)),
            full((H, 3)),
            full((1, 3)),
            full((B, 1)),
        ],
        out_specs=full((B, 1)),
        out_shape=jax.ShapeDtypeStruct((B, 1), jnp.float32),
    )(h, batch_f, ow1, ob1r, ow2p, ob2r, kind_f)


def kernel(protein_pos, protein_atom_feature, ligand_pos, ligand_atom_feature,
           batch_protein, batch_ligand, output_kind,
           Wp, bp, Wl, bl, eW1, eb1, eW2, eb2, infW, infb,
           nW1, nb1, nW2, nb2, oW1, ob1, oW2, ob2):
    npn = protein_pos.shape[0]
    nl = ligand_pos.shape[0]
    n = npn + nl
    dp = protein_atom_feature.shape[1]
    dl = ligand_atom_feature.shape[1]

    br = jnp.arange(B + 1, dtype=jnp.int32)
    sp = jnp.searchsorted(batch_protein, br, side="left").astype(jnp.int32)
    sl = jnp.searchsorted(batch_ligand, br, side="left").astype(jnp.int32)
    off = sp + sl

    q = jnp.arange(N_PAD, dtype=jnp.int32)
    valid = q < n
    bq = jnp.clip(jnp.searchsorted(off, q, side="right").astype(jnp.int32) - 1,
                  0, B - 1)
    npb = sp[bq + 1] - sp[bq]
    rq = q - off[bq]
    perm = jnp.where(rq < npb, sp[bq] + rq, npn + sl[bq] + (rq - npb))
    perm = jnp.where(valid, perm, 0)
    row_lo = jnp.where(valid, off[bq], 0).reshape(1, N_PAD)
    row_hi = jnp.where(valid, off[bq + 1], 0).reshape(1, N_PAD)
    batch_f = jnp.where(valid, bq, B).astype(jnp.float32).reshape(1, N_PAD)

    tstart = jnp.arange(T_KNN, dtype=jnp.int32) * R_KNN
    tlast = jnp.minimum(tstart + R_KNN - 1, n - 1)
    bfirst = bq[tstart]
    blast = bq[tlast]
    clo = (off[bfirst] // CW) * CW
    chi = ((off[blast + 1] + CW - 1) // CW) * CW
    nch = jnp.where(tstart < n, (chi - clo) // CW, 0)
    clo = jnp.where(tstart < n, clo, 0)

    zcol = lambda r, c: jnp.zeros((r, c), jnp.float32)
    prot = jnp.concatenate(
        [protein_atom_feature, zcol(npn, 40 - dp), jnp.ones((npn, 1)),
         zcol(npn, 1), zcol(npn, 6), protein_pos, zcol(npn, 77)], axis=1)
    lig = jnp.concatenate(
        [zcol(nl, dp), ligand_atom_feature, zcol(nl, 40 - dp - dl),
         zcol(nl, 1), jnp.ones((nl, 1)), zcol(nl, 6), ligand_pos,
         zcol(nl, 77)], axis=1)
    fpcomb = jnp.concatenate([prot, lig], axis=0)
    w128 = jnp.concatenate(
        [Wp, Wl, bp[None, :], bl[None, :], jnp.zeros((H - 42, H))],
        axis=0).astype(jnp.float32)

    sortedfp = _sc_gather(fpcomb, perm, window=128)
    posm = sortedfp[:, 48:56]
    pos_t = posm.T

    h = _embed(sortedfp, w128)

    idxkm, dkm = _knn(clo, nch, posm, pos_t, row_lo, row_hi)
    idx_flat = idxkm.reshape(K * N_PAD)
    d3 = dkm[:, :, None]

    half = N_PAD // 2
    idx_a = idxkm[:, :half].reshape(K * half)
    idx_b = idxkm[:, half:].reshape(K * half)
    d3_a, d3_b = d3[:, :half], d3[:, half:]

    for l in range(NUM_LAYERS):
        wargs = (
            eW1[l, :NUM_G, :], eW1[l, NUM_G:NUM_G + H, :],
            eb1[l].reshape(1, H), eW2[l], eb2[l].reshape(1, H),
            infW[l, :, 0].reshape(1, H),
            jnp.broadcast_to(infb[l].reshape(1, 1), (1, H)),
            nW1[l, :H, :], nW1[l, H:, :], nb1[l].reshape(1, H),
            nW2[l], nb2[l].reshape(1, H))
        bv = _proj(h, eW1[l, NUM_G + H:, :])
        ga = _sc_gather(bv, idx_a).reshape(K, half, H)
        gb = _sc_gather(bv, idx_b).reshape(K, half, H)
        ha = _layer(h[:half], d3_a, ga, *wargs)
        hb = _layer(h[half:], d3_b, gb, *wargs)
        h = jnp.concatenate([ha, hb], axis=0)

    return _pool(h, batch_f, oW1, ob1.reshape(1, H), oW2, ob2.reshape(1, 3),
                 output_kind.astype(jnp.float32).reshape(B, 1))
